# Initial kernel scaffold; baseline (speedup 1.0000x reference)
import jax
import jax.numpy as jnp
from jax.experimental import pallas as pl


def kernel(pad_curr, pad_answer, pad_next, emb_table, transform_matrix):
    raise NotImplementedError("write your pallas kernel here")


# speedup vs baseline: 9.3243x; 9.3243x over previous
import functools

import jax
import jax.numpy as jnp
from jax import lax
from jax.experimental import pallas as pl
from jax.experimental.pallas import tpu as pltpu
from jax.experimental.pallas import tpu_sc as plsc

B = 4096
H = 200
D = 64
BT = B * H
NC, NS = 2, 16
NW = NC * NS
SUB = 128
ROWS = BT // SUB
ROWS_W = ROWS // NW
RJ = 2
ITERS = ROWS_W // RJ
CH = RJ * SUB


def _body(curr_hbm, ans_hbm, next_hbm, table_hbm, zeros_hbm,
          inter_out, next_out,
          idx_c0, idx_c1, idx_n0, idx_n1, ans0, ans1,
          rows_c0, rows_c1, rows_n0, rows_n1, zeros_v,
          sd0, sd1, sz0, sz1,
          sem_i0, sem_i1, sem_g0, sem_g1, sem_s0, sem_s1):
  idx_c = (idx_c0, idx_c1)
  idx_n = (idx_n0, idx_n1)
  ans_v = (ans0, ans1)
  rows_c = (rows_c0, rows_c1)
  rows_n = (rows_n0, rows_n1)
  sidx_d = (sd0, sd1)
  sidx_z = (sz0, sz1)
  sem_i = (sem_i0, sem_i1)
  sem_g = (sem_g0, sem_g1)
  sem_s = (sem_s0, sem_s1)

  wid = lax.axis_index("s") * NC + lax.axis_index("c")
  wbase = wid * ROWS_W

  pltpu.sync_copy(zeros_hbm, zeros_v)

  def issue_loads(b, t):
    r0 = jnp.minimum(wbase + t * RJ, ROWS - RJ)
    pltpu.async_copy(curr_hbm.at[pl.ds(r0, RJ)], idx_c[b], sem_i[b])
    pltpu.async_copy(next_hbm.at[pl.ds(r0, RJ)], idx_n[b], sem_i[b])
    pltpu.async_copy(ans_hbm.at[pl.ds(r0, RJ)], ans_v[b], sem_i[b])

  def drain_loads(b):
    pltpu.make_async_copy(curr_hbm.at[pl.ds(0, RJ)], idx_c[b], sem_i[b]).wait()
    pltpu.make_async_copy(next_hbm.at[pl.ds(0, RJ)], idx_n[b], sem_i[b]).wait()
    pltpu.make_async_copy(ans_hbm.at[pl.ds(0, RJ)], ans_v[b], sem_i[b]).wait()

  def issue_gathers(b):
    gs = []
    for j in range(RJ):
      gs.append(pltpu.async_copy(table_hbm.at[idx_c[b].at[j]],
                                 rows_c[b].at[pl.ds(j * SUB, SUB)], sem_g[b]))
      gs.append(pltpu.async_copy(table_hbm.at[idx_n[b].at[j]],
                                 rows_n[b].at[pl.ds(j * SUB, SUB)], sem_g[b]))
    return gs

  def compute_sidx(b, t):
    r0 = wbase + t * RJ
    iota = lax.iota(jnp.int32, 16)
    for j in range(RJ):
      base2 = (r0 + j) * (2 * SUB)
      for k in range(SUB // 16):
        a = ans_v[b][j, pl.ds(k * 16, 16)]
        p2 = base2 + (k * 32) + 2 * iota
        sidx_d[b][j, pl.ds(k * 16, 16)] = p2 + 1 - a
        sidx_z[b][j, pl.ds(k * 16, 16)] = p2 + a

  def issue_scatters(b, t):
    r0 = wbase + t * RJ
    for j in range(RJ):
      pltpu.async_copy(rows_c[b].at[pl.ds(j * SUB, SUB)],
                       inter_out.at[sidx_d[b].at[j]], sem_s[b])
      pltpu.async_copy(zeros_v, inter_out.at[sidx_z[b].at[j]], sem_s[b])
    pltpu.async_copy(rows_n[b], next_out.at[pl.ds(r0 * SUB, CH)], sem_s[b])

  def drain_scatters(b):
    for j in range(RJ):
      pltpu.make_async_copy(rows_c[b].at[pl.ds(j * SUB, SUB)],
                            inter_out.at[sidx_d[b].at[j]], sem_s[b]).wait()
      pltpu.make_async_copy(zeros_v, inter_out.at[sidx_z[b].at[j]],
                            sem_s[b]).wait()
    pltpu.make_async_copy(rows_n[b], next_out.at[pl.ds(0, CH)], sem_s[b]).wait()

  def iteration(b, t, first):
    if not first:
      drain_scatters(b)
    drain_loads(b)
    gs = issue_gathers(b)
    issue_loads(b ^ 1, t + 1)
    compute_sidx(b, t)
    for g in gs:
      g.wait()
    issue_scatters(b, t)

  issue_loads(0, 0)
  iteration(0, 0, True)
  iteration(1, 1, True)

  def step(s, carry):
    iteration(0, 2 * s, False)
    iteration(1, 2 * s + 1, False)
    return carry

  lax.fori_loop(1, ITERS // 2, step, 0)

  drain_scatters(0)
  drain_scatters(1)
  drain_loads(ITERS % 2)


@jax.jit
def _run(curr2, ans2, next2, table, zeros):
  k = pl.kernel(
      _body,
      out_type=(
          jax.ShapeDtypeStruct((2 * BT, D), jnp.float32),
          jax.ShapeDtypeStruct((BT, D), jnp.float32),
      ),
      mesh=plsc.VectorSubcoreMesh(core_axis_name="c", subcore_axis_name="s"),
      scratch_types=[
          pltpu.VMEM((RJ, SUB), jnp.int32),
          pltpu.VMEM((RJ, SUB), jnp.int32),
          pltpu.VMEM((RJ, SUB), jnp.int32),
          pltpu.VMEM((RJ, SUB), jnp.int32),
          pltpu.VMEM((RJ, SUB), jnp.int32),
          pltpu.VMEM((RJ, SUB), jnp.int32),
          pltpu.VMEM((CH, D), jnp.float32),
          pltpu.VMEM((CH, D), jnp.float32),
          pltpu.VMEM((CH, D), jnp.float32),
          pltpu.VMEM((CH, D), jnp.float32),
          pltpu.VMEM((SUB, D), jnp.float32),
          pltpu.VMEM((RJ, SUB), jnp.int32),
          pltpu.VMEM((RJ, SUB), jnp.int32),
          pltpu.VMEM((RJ, SUB), jnp.int32),
          pltpu.VMEM((RJ, SUB), jnp.int32),
          pltpu.SemaphoreType.DMA,
          pltpu.SemaphoreType.DMA,
          pltpu.SemaphoreType.DMA,
          pltpu.SemaphoreType.DMA,
          pltpu.SemaphoreType.DMA,
          pltpu.SemaphoreType.DMA,
      ],
      compiler_params=pltpu.CompilerParams(use_tc_tiling_on_sc=False),
  )
  return k(curr2, ans2, next2, table, zeros)


def kernel(pad_curr, pad_answer, pad_next, emb_table, transform_matrix):
  curr2 = pad_curr.reshape(ROWS, SUB).astype(jnp.int32)
  ans2 = pad_answer.reshape(ROWS, SUB).astype(jnp.int32)
  next2 = pad_next.reshape(ROWS, SUB).astype(jnp.int32)
  zeros = jnp.zeros((SUB, D), jnp.float32)
  inter2, nxt = _run(curr2, ans2, next2, emb_table, zeros)
  return inter2.reshape(B, H, 2 * D), nxt.reshape(B, H, D)
